# baseline (speedup 1.0000x reference)
import functools

import jax
import jax.numpy as jnp
from jax import lax
from jax.experimental import pallas as pl
from jax.experimental.pallas import tpu as pltpu

_N = 8192
_M = 1024
_D = 256
_B = 8
_HW = 1024
_MB = 256
_BSTEPS = _M // _MB


def _fused(q_ref, keysf_ref, keysb_ref, w_ref, gamma_ref, beta_ref,
           sm_ref, sq_ref, uq_ref, gl_ref, sl_ref,
           qn_s, conv_s, bnsum_s, bnsq_s, gp_s, sp_s, kst_s):
    f32 = jnp.float32
    i = pl.program_id(0)

    @pl.when(i == 0)
    def _key_stats():
        hi = jax.lax.Precision.HIGHEST
        eps = 1e-6
        keys = keysf_ref[...]
        ones_d = jnp.ones((1, _D), f32)
        kst_s[0:1, :] = lax.dot_general(
            ones_d, keys * (keys - 2.0 * eps), (((1,), (1,)), ((), ())),
            precision=hi, preferred_element_type=f32)

    @pl.when(i < _B)
    def _phase_a():
        q = q_ref[...].reshape(_D, _HW)
        n2 = jnp.sum(q * q, axis=0, keepdims=True)
        denom = jnp.maximum(jnp.sqrt(n2), 1e-12)
        qn = q / denom
        qn_s[i] = qn
        qsq_l = n2 / (denom * denom)
        qsum_l = jnp.sum(q, axis=0, keepdims=True) / denom

        eps = 1e-6
        a_l = qsq_l + (2.0 * eps) * qsum_l + _D * eps * eps
        a = jnp.transpose(a_l, (1, 0))

        keys = keysf_ref[...]
        s = lax.dot_general(qn, keys, (((0,), (1,)), ((), ())),
                            preferred_element_type=f32)

        m1 = jnp.max(s, axis=1, keepdims=True)
        e = jnp.exp(s - m1)
        p = e * (1.0 / jnp.sum(e, axis=1, keepdims=True))
        sm_ref[...] = p

        g_t = kst_s[0:1, :]

        mk1 = s == m1
        kv1 = jnp.max(jnp.where(mk1, g_t, -jnp.inf), axis=1, keepdims=True)
        s2 = jnp.where(mk1, -jnp.inf, s)
        m2 = jnp.max(s2, axis=1, keepdims=True)
        kv2 = jnp.max(jnp.where(s2 == m2, g_t, -jnp.inf), axis=1,
                      keepdims=True)

        dpos2 = jnp.maximum(a - 2.0 * m1 + kv1, 0.0)
        dpos = jnp.sqrt(dpos2)
        dneg = jnp.sqrt(jnp.maximum(a - 2.0 * m2 + kv2, 0.0))
        gp = jnp.sum(dpos2)
        sp = jnp.sum(jnp.maximum(dpos - dneg + 1.0, 0.0))
        gp_part = jnp.full((1, 128), gp, f32)
        sp_part = jnp.full((1, 128), sp, f32)
        gp_s[...] = jnp.where(i == 0, gp_part, gp_s[...] + gp_part)
        sp_s[...] = jnp.where(i == 0, sp_part, sp_s[...] + sp_part)

        c_t = lax.dot_general(keys, p, (((0,), (1,)), ((), ())),
                              preferred_element_type=f32)
        w1 = w_ref[0:_D, :]
        w2 = w_ref[_D:2 * _D, :]
        conv = (lax.dot_general(w1, qn, (((0,), (0,)), ((), ())),
                                preferred_element_type=f32) +
                lax.dot_general(w2, c_t, (((0,), (0,)), ((), ())),
                                preferred_element_type=f32))
        conv_s[i] = conv
        csum = jnp.sum(conv, axis=1, keepdims=True)
        csq = jnp.sum(conv * conv, axis=1, keepdims=True)
        bnsum_s[...] = jnp.where(i == 0, csum, bnsum_s[...] + csum)
        bnsq_s[...] = jnp.where(i == 0, csq, bnsq_s[...] + csq)

    @pl.when(i >= _B)
    def _phase_b():
        kb = keysb_ref[...]
        sb = [lax.dot_general(qn_s[b], kb, (((0,), (1,)), ((), ())),
                              preferred_element_type=f32)
              for b in range(_B)]
        cm = sb[0].max(axis=0, keepdims=True)
        for x in sb[1:]:
            cm = jnp.maximum(cm, x.max(axis=0, keepdims=True))
        eb = [jnp.exp(x - cm) for x in sb]
        cs = eb[0].sum(axis=0, keepdims=True)
        for x in eb[1:]:
            cs = cs + x.sum(axis=0, keepdims=True)
        rcs = 1.0 / cs
        for b in range(_B):
            sq_ref[b * _HW:(b + 1) * _HW, :] = eb[b] * rcs

        mean = bnsum_s[...] * (1.0 / _N)
        var = bnsq_s[...] * (1.0 / _N) - mean * mean
        inv = 1.0 / jnp.sqrt(var + 1e-5)
        scale = inv * gamma_ref[...]
        for t in range(2):
            conv = conv_s[2 * (i - _B) + t]
            y = jnp.maximum((conv - mean) * scale + beta_ref[...], 0.0)
            uq_ref[t:t + 1] = y.reshape(1, _D, _HW)

        gl_ref[...] = gp_s[0:1, 0:1] * (1.0 / (_N * _D))
        sl_ref[...] = sp_s[0:1, 0:1] * (1.0 / _N)


@functools.partial(jax.jit, static_argnames=())
def kernel(query, keys, W, gamma, beta):
    f32 = jnp.float32
    q3 = query.reshape(_B, _D, _HW)
    sm, sq, uq, gl, sl = pl.pallas_call(
        _fused,
        grid=(_B + _BSTEPS,),
        in_specs=[
            pl.BlockSpec((1, _D, _HW), lambda i: (jnp.minimum(i, _B - 1), 0, 0)),
            pl.BlockSpec((_M, _D), lambda i: (0, 0)),
            pl.BlockSpec((_MB, _D), lambda i: (jnp.maximum(i - _B, 0), 0)),
            pl.BlockSpec((2 * _D, _D), lambda i: (0, 0)),
            pl.BlockSpec((_D, 1), lambda i: (0, 0)),
            pl.BlockSpec((_D, 1), lambda i: (0, 0)),
        ],
        out_specs=[
            pl.BlockSpec((_HW, _M), lambda i: (jnp.minimum(i, _B - 1), 0)),
            pl.BlockSpec((_N, _MB), lambda i: (0, jnp.maximum(i - _B, 0))),
            pl.BlockSpec((2, _D, _HW), lambda i: (jnp.maximum(i - _B, 0), 0, 0)),
            pl.BlockSpec((1, 1), lambda i: (0, 0)),
            pl.BlockSpec((1, 1), lambda i: (0, 0)),
        ],
        out_shape=[
            jax.ShapeDtypeStruct((_N, _M), f32),
            jax.ShapeDtypeStruct((_N, _M), f32),
            jax.ShapeDtypeStruct((_B, _D, _HW), f32),
            jax.ShapeDtypeStruct((1, 1), f32),
            jax.ShapeDtypeStruct((1, 1), f32),
        ],
        scratch_shapes=[
            pltpu.VMEM((_B, _D, _HW), f32),
            pltpu.VMEM((_B, _D, _HW), f32),
            pltpu.VMEM((_D, 1), f32),
            pltpu.VMEM((_D, 1), f32),
            pltpu.VMEM((1, 128), f32),
            pltpu.VMEM((1, 128), f32),
            pltpu.VMEM((1, _M), f32),
        ],
    )(q3, keys, keys, W, gamma.reshape(_D, 1), beta.reshape(_D, 1))

    return (uq.reshape(_B, _D, 32, 32), sq, sm, gl.reshape(()), sl.reshape(()))


# speedup vs baseline: 1.0002x; 1.0002x over previous
import functools

import jax
import jax.numpy as jnp
from jax import lax
from jax.experimental import pallas as pl
from jax.experimental.pallas import tpu as pltpu

_N = 8192
_M = 1024
_D = 256
_B = 8
_HW = 1024
_MB = 256
_BSTEPS = _M // _MB


def _fused(q_ref, keysf_ref, keysb_ref, w_ref, gamma_ref, beta_ref,
           sm_ref, sq_ref, uq_ref, gl_ref, sl_ref,
           qn_s, conv_s, bnsum_s, bnsq_s, gp_s, sp_s, kst_s):
    f32 = jnp.float32
    i = pl.program_id(0)

    @pl.when(i == 0)
    def _key_stats():
        hi = jax.lax.Precision.HIGHEST
        eps = 1e-6
        keys = keysf_ref[...]
        ones_d = jnp.ones((1, _D), f32)
        kst_s[0:1, :] = lax.dot_general(
            ones_d, keys * (keys - 2.0 * eps), (((1,), (1,)), ((), ())),
            precision=hi, preferred_element_type=f32)

    @pl.when(i < _B)
    def _phase_a():
        q = q_ref[...].reshape(_D, _HW)
        n2 = jnp.sum(q * q, axis=0, keepdims=True)
        denom = jnp.maximum(jnp.sqrt(n2), 1e-12)
        qn = q / denom
        qn_s[i] = qn
        eps = 1e-6
        a = 1.0 + _D * eps * eps

        keys = keysf_ref[...]
        s = lax.dot_general(qn, keys, (((0,), (1,)), ((), ())),
                            preferred_element_type=f32)

        m1 = jnp.max(s, axis=1, keepdims=True)
        e = jnp.exp(s - m1)
        p = e * (1.0 / jnp.sum(e, axis=1, keepdims=True))
        sm_ref[...] = p

        g_t = kst_s[0:1, :]

        mk1 = s == m1
        kv1 = jnp.max(jnp.where(mk1, g_t, -jnp.inf), axis=1, keepdims=True)
        s2 = jnp.where(mk1, -jnp.inf, s)
        m2 = jnp.max(s2, axis=1, keepdims=True)
        kv2 = jnp.max(jnp.where(s2 == m2, g_t, -jnp.inf), axis=1,
                      keepdims=True)

        dpos2 = jnp.maximum(a - 2.0 * m1 + kv1, 0.0)
        dpos = jnp.sqrt(dpos2)
        dneg = jnp.sqrt(jnp.maximum(a - 2.0 * m2 + kv2, 0.0))
        gp = jnp.sum(dpos2)
        sp = jnp.sum(jnp.maximum(dpos - dneg + 1.0, 0.0))
        gp_part = jnp.full((1, 128), gp, f32)
        sp_part = jnp.full((1, 128), sp, f32)
        gp_s[...] = jnp.where(i == 0, gp_part, gp_s[...] + gp_part)
        sp_s[...] = jnp.where(i == 0, sp_part, sp_s[...] + sp_part)

        c_t = lax.dot_general(keys, p, (((0,), (1,)), ((), ())),
                              preferred_element_type=f32)
        w1 = w_ref[0:_D, :]
        w2 = w_ref[_D:2 * _D, :]
        conv = (lax.dot_general(w1, qn, (((0,), (0,)), ((), ())),
                                preferred_element_type=f32) +
                lax.dot_general(w2, c_t, (((0,), (0,)), ((), ())),
                                preferred_element_type=f32))
        conv_s[i] = conv
        csum = jnp.sum(conv, axis=1, keepdims=True)
        csq = jnp.sum(conv * conv, axis=1, keepdims=True)
        bnsum_s[...] = jnp.where(i == 0, csum, bnsum_s[...] + csum)
        bnsq_s[...] = jnp.where(i == 0, csq, bnsq_s[...] + csq)

    @pl.when(i >= _B)
    def _phase_b():
        kb = keysb_ref[...]
        sb = [lax.dot_general(qn_s[b], kb, (((0,), (1,)), ((), ())),
                              preferred_element_type=f32)
              for b in range(_B)]
        cm = sb[0].max(axis=0, keepdims=True)
        for x in sb[1:]:
            cm = jnp.maximum(cm, x.max(axis=0, keepdims=True))
        eb = [jnp.exp(x - cm) for x in sb]
        cs = eb[0].sum(axis=0, keepdims=True)
        for x in eb[1:]:
            cs = cs + x.sum(axis=0, keepdims=True)
        rcs = 1.0 / cs
        for b in range(_B):
            sq_ref[b * _HW:(b + 1) * _HW, :] = eb[b] * rcs

        mean = bnsum_s[...] * (1.0 / _N)
        var = bnsq_s[...] * (1.0 / _N) - mean * mean
        inv = 1.0 / jnp.sqrt(var + 1e-5)
        scale = inv * gamma_ref[...]
        for t in range(2):
            conv = conv_s[2 * (i - _B) + t]
            y = jnp.maximum((conv - mean) * scale + beta_ref[...], 0.0)
            uq_ref[t:t + 1] = y.reshape(1, _D, _HW)

        gl_ref[...] = gp_s[0:1, 0:1] * (1.0 / (_N * _D))
        sl_ref[...] = sp_s[0:1, 0:1] * (1.0 / _N)


@functools.partial(jax.jit, static_argnames=())
def kernel(query, keys, W, gamma, beta):
    f32 = jnp.float32
    q3 = query.reshape(_B, _D, _HW)
    sm, sq, uq, gl, sl = pl.pallas_call(
        _fused,
        grid=(_B + _BSTEPS,),
        in_specs=[
            pl.BlockSpec((1, _D, _HW), lambda i: (jnp.minimum(i, _B - 1), 0, 0)),
            pl.BlockSpec((_M, _D), lambda i: (0, 0)),
            pl.BlockSpec((_MB, _D), lambda i: (jnp.maximum(i - _B, 0), 0)),
            pl.BlockSpec((2 * _D, _D), lambda i: (0, 0)),
            pl.BlockSpec((_D, 1), lambda i: (0, 0)),
            pl.BlockSpec((_D, 1), lambda i: (0, 0)),
        ],
        out_specs=[
            pl.BlockSpec((_HW, _M), lambda i: (jnp.minimum(i, _B - 1), 0)),
            pl.BlockSpec((_N, _MB), lambda i: (0, jnp.maximum(i - _B, 0))),
            pl.BlockSpec((2, _D, _HW), lambda i: (jnp.maximum(i - _B, 0), 0, 0)),
            pl.BlockSpec((1, 1), lambda i: (0, 0)),
            pl.BlockSpec((1, 1), lambda i: (0, 0)),
        ],
        out_shape=[
            jax.ShapeDtypeStruct((_N, _M), f32),
            jax.ShapeDtypeStruct((_N, _M), f32),
            jax.ShapeDtypeStruct((_B, _D, _HW), f32),
            jax.ShapeDtypeStruct((1, 1), f32),
            jax.ShapeDtypeStruct((1, 1), f32),
        ],
        scratch_shapes=[
            pltpu.VMEM((_B, _D, _HW), f32),
            pltpu.VMEM((_B, _D, _HW), f32),
            pltpu.VMEM((_D, 1), f32),
            pltpu.VMEM((_D, 1), f32),
            pltpu.VMEM((1, 128), f32),
            pltpu.VMEM((1, 128), f32),
            pltpu.VMEM((1, _M), f32),
        ],
    )(q3, keys, keys, W, gamma.reshape(_D, 1), beta.reshape(_D, 1))

    return (uq.reshape(_B, _D, 32, 32), sq, sm, gl.reshape(()), sl.reshape(()))
